# baseline (speedup 1.0000x reference)
import functools

import jax
import jax.numpy as jnp
from jax import lax
from jax.experimental import pallas as pl
from jax.experimental.pallas import tpu as pltpu
from jax.experimental.pallas import tpu_sc as plsc


def _pool_body(x_hbm, emb_hbm, out_hbm, idx_v, rows_v, pool_v, sem,
               *, rows_per_w, L, EMB, chunk, NC):
    wid = lax.axis_index("s") * NC + lax.axis_index("c")
    n_idx = rows_per_w * L
    pltpu.sync_copy(x_hbm.at[pl.ds(wid * n_idx, n_idx)], idx_v)

    nchunks = rows_per_w // chunk
    nvec = EMB // 16

    def body(c, carry):
        cp = pltpu.async_copy(
            emb_hbm.at[idx_v.at[pl.ds(c * (chunk * L), chunk * L)]],
            rows_v, sem)
        cp.wait()
        for i in range(chunk):
            accs = [jnp.zeros((16,), jnp.float32) for _ in range(nvec)]
            for l in range(L):
                for j in range(nvec):
                    accs[j] = accs[j] + rows_v[i * L + l, pl.ds(j * 16, 16)]
            for j in range(nvec):
                pool_v[c * chunk + i, pl.ds(j * 16, 16)] = (
                    accs[j] * (1.0 / L))
        return carry

    lax.fori_loop(0, nchunks, body, 0)
    pltpu.sync_copy(pool_v, out_hbm.at[pl.ds(wid * rows_per_w, rows_per_w), :])


def _sc_pool(x_flat, emb, B, L, EMB):
    info = plsc.get_sparse_core_info()
    NC, NS = info.num_cores, info.num_subcores
    NW = NC * NS
    rows_per_w = B // NW
    chunk = 4
    mesh = plsc.VectorSubcoreMesh(core_axis_name="c", subcore_axis_name="s")
    body = functools.partial(_pool_body, rows_per_w=rows_per_w, L=L,
                             EMB=EMB, chunk=chunk, NC=NC)
    return pl.kernel(
        body,
        out_type=jax.ShapeDtypeStruct((B, EMB), jnp.float32),
        mesh=mesh,
        compiler_params=pltpu.CompilerParams(use_tc_tiling_on_sc=False),
        scratch_types=[
            pltpu.VMEM((rows_per_w * L,), jnp.int32),
            pltpu.VMEM((chunk * L, EMB), jnp.float32),
            pltpu.VMEM((rows_per_w, EMB), jnp.float32),
            pltpu.SemaphoreType.DMA,
        ],
    )(x_flat, emb)


def _mlp_body(p_ref, W1_ref, b1_ref, W2_ref, b2_ref, o_ref):
    h = jnp.dot(p_ref[...], W1_ref[...],
                preferred_element_type=jnp.float32) + b1_ref[...]
    h = jnp.maximum(h, 0.0)
    o_ref[...] = jnp.dot(h, W2_ref[...],
                         preferred_element_type=jnp.float32) + b2_ref[...]


def _tc_mlp(pooled, W1, b1, W2, b2):
    B = pooled.shape[0]
    return pl.pallas_call(
        _mlp_body,
        out_shape=jax.ShapeDtypeStruct((B, W2.shape[1]), jnp.float32),
    )(pooled, W1, b1, W2, b2)


@jax.jit
def kernel(x, emb, W1, b1, W2, b2):
    B, L = x.shape
    EMB = emb.shape[1]
    x_flat = x.reshape(-1).astype(jnp.int32)
    pooled = _sc_pool(x_flat, emb, B, L, EMB)
    return _tc_mlp(pooled, W1, b1.reshape(1, -1), W2, b2.reshape(1, -1))


# speedup vs baseline: 1.8383x; 1.8383x over previous
import functools

import jax
import jax.numpy as jnp
from jax import lax
from jax.experimental import pallas as pl
from jax.experimental.pallas import tpu as pltpu
from jax.experimental.pallas import tpu_sc as plsc


def _tr_body(embT_ref, o_ref):
    o_ref[:, 0:embT_ref.shape[0]] = embT_ref[...].T


def _tc_widen(embT, pad_to=128, blk=8192):
    EMB, V = embT.shape
    grid = (V + blk - 1) // blk
    return pl.pallas_call(
        _tr_body,
        grid=(grid,),
        in_specs=[pl.BlockSpec((EMB, blk), lambda i: (0, i))],
        out_specs=pl.BlockSpec((blk, pad_to), lambda i: (i, 0)),
        out_shape=jax.ShapeDtypeStruct((V, pad_to), jnp.float32),
    )(embT)


def _pool_body(x_hbm, emb_hbm, out_hbm, idx_v, rows_v, pool_v, sem,
               *, rows_per_w, L, EMB, chunk, NC):
    wid = lax.axis_index("s") * NC + lax.axis_index("c")
    n_idx = rows_per_w * L
    pltpu.sync_copy(x_hbm.at[pl.ds(wid * n_idx, n_idx)], idx_v)

    nchunks = rows_per_w // chunk
    nvec = EMB // 16

    def body(c, carry):
        cp = pltpu.async_copy(
            emb_hbm.at[idx_v.at[pl.ds(c * (chunk * L), chunk * L)]],
            rows_v, sem)
        cp.wait()
        for i in range(chunk):
            accs = [jnp.zeros((16,), jnp.float32) for _ in range(nvec)]
            for l in range(L):
                for j in range(nvec):
                    accs[j] = accs[j] + rows_v[i * L + l, pl.ds(j * 16, 16)]
            for j in range(nvec):
                pool_v[c * chunk + i, pl.ds(j * 16, 16)] = (
                    accs[j] * (1.0 / L))
        return carry

    lax.fori_loop(0, nchunks, body, 0)
    pltpu.sync_copy(pool_v, out_hbm.at[pl.ds(wid * rows_per_w, rows_per_w), :])


def _sc_pool(x_flat, emb_wide, B, L, EMB):
    info = plsc.get_sparse_core_info()
    NC, NS = info.num_cores, info.num_subcores
    NW = NC * NS
    rows_per_w = B // NW
    chunk = 4
    mesh = plsc.VectorSubcoreMesh(core_axis_name="c", subcore_axis_name="s")
    body = functools.partial(_pool_body, rows_per_w=rows_per_w, L=L,
                             EMB=EMB, chunk=chunk, NC=NC)
    return pl.kernel(
        body,
        out_type=jax.ShapeDtypeStruct((B, EMB), jnp.float32),
        mesh=mesh,
        compiler_params=pltpu.CompilerParams(use_tc_tiling_on_sc=True),
        scratch_types=[
            pltpu.VMEM((rows_per_w * L,), jnp.int32),
            pltpu.VMEM((chunk * L, 128), jnp.float32),
            pltpu.VMEM((rows_per_w, EMB), jnp.float32),
            pltpu.SemaphoreType.DMA,
        ],
    )(x_flat, emb_wide)


def _mlp_body(p_ref, W1_ref, b1_ref, W2_ref, b2_ref, o_ref):
    h = jnp.dot(p_ref[...], W1_ref[...],
                preferred_element_type=jnp.float32) + b1_ref[...]
    h = jnp.maximum(h, 0.0)
    o_ref[...] = jnp.dot(h, W2_ref[...],
                         preferred_element_type=jnp.float32) + b2_ref[...]


def _tc_mlp(pooled, W1, b1, W2, b2):
    B = pooled.shape[0]
    return pl.pallas_call(
        _mlp_body,
        out_shape=jax.ShapeDtypeStruct((B, W2.shape[1]), jnp.float32),
    )(pooled, W1, b1.reshape(1, -1), W2, b2.reshape(1, -1))


@jax.jit
def kernel(x, emb, W1, b1, W2, b2):
    B, L = x.shape
    EMB = emb.shape[1]
    x_flat = x.reshape(-1).astype(jnp.int32)
    emb_wide = _tc_widen(jnp.swapaxes(emb, 0, 1))
    pooled = _sc_pool(x_flat, emb_wide, B, L, EMB)
    return _tc_mlp(pooled, W1, b1.reshape(1, -1), W2, b2.reshape(1, -1))


# speedup vs baseline: 1.9400x; 1.0553x over previous
import functools

import jax
import jax.numpy as jnp
from jax import lax
from jax.experimental import pallas as pl
from jax.experimental.pallas import tpu as pltpu
from jax.experimental.pallas import tpu_sc as plsc


def _tr_body(embT_ref, o_ref):
    o_ref[:, 0:embT_ref.shape[0]] = embT_ref[...].T


def _tc_widen(embT, pad_to=128, blk=8192):
    EMB, V = embT.shape
    grid = (V + blk - 1) // blk
    return pl.pallas_call(
        _tr_body,
        grid=(grid,),
        in_specs=[pl.BlockSpec((EMB, blk), lambda i: (0, i))],
        out_specs=pl.BlockSpec((blk, pad_to), lambda i: (i, 0)),
        out_shape=jax.ShapeDtypeStruct((V, pad_to), jnp.float32),
    )(embT)


def _pool_body(x_hbm, emb_hbm, out_hbm, idx_v, rows_v0, rows_v1, pool_v,
               sem0, sem1, *, rows_per_w, L, EMB, chunk, NC):
    wid = lax.axis_index("s") * NC + lax.axis_index("c")
    n_idx = rows_per_w * L
    pltpu.sync_copy(x_hbm.at[pl.ds(wid * n_idx, n_idx)], idx_v)

    cl = chunk * L
    nchunks = rows_per_w // chunk
    nvec = EMB // 16

    def fire(c, buf, sem):
        pltpu.async_copy(emb_hbm.at[idx_v.at[pl.ds(c * cl, cl)]], buf, sem)

    def drain(c, buf, sem):
        pltpu.make_async_copy(
            emb_hbm.at[idx_v.at[pl.ds(c * cl, cl)]], buf, sem).wait()

    def accumulate(c, buf):
        for i in range(chunk):
            accs = [jnp.zeros((16,), jnp.float32) for _ in range(nvec)]
            for l in range(L):
                for j in range(nvec):
                    accs[j] = accs[j] + buf[i * L + l, pl.ds(j * 16, 16)]
            for j in range(nvec):
                pool_v[c * chunk + i, pl.ds(j * 16, 16)] = (
                    accs[j] * (1.0 / L))

    fire(0, rows_v0, sem0)
    fire(1, rows_v1, sem1)

    def body(g, carry):
        c0 = 2 * g
        drain(c0, rows_v0, sem0)
        accumulate(c0, rows_v0)

        @pl.when(c0 + 2 < nchunks)
        def _():
            fire(c0 + 2, rows_v0, sem0)

        c1 = c0 + 1
        drain(c1, rows_v1, sem1)
        accumulate(c1, rows_v1)

        @pl.when(c1 + 2 < nchunks)
        def _():
            fire(c1 + 2, rows_v1, sem1)

        return carry

    lax.fori_loop(0, nchunks // 2, body, 0)
    pltpu.sync_copy(pool_v, out_hbm.at[pl.ds(wid * rows_per_w, rows_per_w), :])


def _sc_pool(x_flat, emb_wide, B, L, EMB):
    info = plsc.get_sparse_core_info()
    NC, NS = info.num_cores, info.num_subcores
    NW = NC * NS
    rows_per_w = B // NW
    chunk = 4
    mesh = plsc.VectorSubcoreMesh(core_axis_name="c", subcore_axis_name="s")
    body = functools.partial(_pool_body, rows_per_w=rows_per_w, L=L,
                             EMB=EMB, chunk=chunk, NC=NC)
    return pl.kernel(
        body,
        out_type=jax.ShapeDtypeStruct((B, EMB), jnp.float32),
        mesh=mesh,
        compiler_params=pltpu.CompilerParams(use_tc_tiling_on_sc=True),
        scratch_types=[
            pltpu.VMEM((rows_per_w * L,), jnp.int32),
            pltpu.VMEM((chunk * L, 128), jnp.float32),
            pltpu.VMEM((chunk * L, 128), jnp.float32),
            pltpu.VMEM((rows_per_w, EMB), jnp.float32),
            pltpu.SemaphoreType.DMA,
            pltpu.SemaphoreType.DMA,
        ],
    )(x_flat, emb_wide)


def _mlp_body(p_ref, W1_ref, b1_ref, W2_ref, b2_ref, o_ref):
    h = jnp.dot(p_ref[...], W1_ref[...],
                preferred_element_type=jnp.float32) + b1_ref[...]
    h = jnp.maximum(h, 0.0)
    o_ref[...] = jnp.dot(h, W2_ref[...],
                         preferred_element_type=jnp.float32) + b2_ref[...]


def _tc_mlp(pooled, W1, b1, W2, b2):
    B = pooled.shape[0]
    return pl.pallas_call(
        _mlp_body,
        out_shape=jax.ShapeDtypeStruct((B, W2.shape[1]), jnp.float32),
    )(pooled, W1, b1.reshape(1, -1), W2, b2.reshape(1, -1))


@jax.jit
def kernel(x, emb, W1, b1, W2, b2):
    B, L = x.shape
    EMB = emb.shape[1]
    x_flat = x.reshape(-1).astype(jnp.int32)
    emb_wide = _tc_widen(jnp.swapaxes(emb, 0, 1))
    pooled = _sc_pool(x_flat, emb_wide, B, L, EMB)
    return _tc_mlp(pooled, W1, b1.reshape(1, -1), W2, b2.reshape(1, -1))


# speedup vs baseline: 1.9786x; 1.0199x over previous
import functools

import jax
import jax.numpy as jnp
from jax import lax
from jax.experimental import pallas as pl
from jax.experimental.pallas import tpu as pltpu
from jax.experimental.pallas import tpu_sc as plsc


def _tr2_body(lo_ref, hi_ref, o_ref):
    EMB = lo_ref.shape[0]
    o_ref[:, 0:EMB] = lo_ref[...].T
    o_ref[:, EMB:2 * EMB] = hi_ref[...].T


def _tc_pack(embT, blk=8192):
    EMB, V = embT.shape
    nblk_half = (V // 2 + blk - 1) // blk
    H = nblk_half * blk
    max_blk = (V - 1) // blk
    return pl.pallas_call(
        _tr2_body,
        grid=(nblk_half,),
        in_specs=[pl.BlockSpec((EMB, blk), lambda i: (0, i)),
                  pl.BlockSpec((EMB, blk),
                               lambda i: (0, jnp.minimum(i + nblk_half,
                                                         max_blk)))],
        out_specs=pl.BlockSpec((blk, 2 * EMB), lambda i: (i, 0)),
        out_shape=jax.ShapeDtypeStruct((H, 2 * EMB), jnp.float32),
    )(embT, embT), H


def _pool_body(x_hbm, emb_hbm, out_hbm, idx_v, offs_v, rows_v0, rows_v1,
               pool_v, sem0, sem1, *, rows_per_w, L, EMB, chunk, NC, H):
    wid = lax.axis_index("s") * NC + lax.axis_index("c")
    n_idx = rows_per_w * L
    pltpu.sync_copy(x_hbm.at[pl.ds(wid * n_idx, n_idx)], idx_v)

    def prep(k, carry):
        v = idx_v[pl.ds(k * 16, 16)]
        hi = v >= H
        idx_v[pl.ds(k * 16, 16)] = jnp.where(hi, v - H, v)
        offs_v[pl.ds(k * 16, 16)] = jnp.where(hi, EMB, 0).astype(jnp.int32)
        return carry

    lax.fori_loop(0, n_idx // 16, prep, 0)

    cl = chunk * L
    nchunks = rows_per_w // chunk
    nvec = EMB // 16
    iotas = [lax.iota(jnp.int32, 16) + j * 16 for j in range(nvec)]

    def fire(c, buf, sem):
        pltpu.async_copy(emb_hbm.at[idx_v.at[pl.ds(c * cl, cl)]], buf, sem)

    def drain(c, buf, sem):
        pltpu.make_async_copy(
            emb_hbm.at[idx_v.at[pl.ds(c * cl, cl)]], buf, sem).wait()

    def accumulate(c, buf):
        for i in range(chunk):
            accs = [jnp.zeros((16,), jnp.float32) for _ in range(nvec)]
            for l in range(L):
                s = i * L + l
                off = plsc.load_gather(
                    offs_v, [jnp.full((16,), c * cl + s, jnp.int32)])
                row = jnp.full((16,), s, jnp.int32)
                for j in range(nvec):
                    accs[j] = accs[j] + plsc.load_gather(
                        buf, [row, off + iotas[j]])
            for j in range(nvec):
                pool_v[c * chunk + i, pl.ds(j * 16, 16)] = (
                    accs[j] * (1.0 / L))

    fire(0, rows_v0, sem0)
    fire(1, rows_v1, sem1)

    def body(g, carry):
        c0 = 2 * g
        drain(c0, rows_v0, sem0)
        accumulate(c0, rows_v0)

        @pl.when(c0 + 2 < nchunks)
        def _():
            fire(c0 + 2, rows_v0, sem0)

        c1 = c0 + 1
        drain(c1, rows_v1, sem1)
        accumulate(c1, rows_v1)

        @pl.when(c1 + 2 < nchunks)
        def _():
            fire(c1 + 2, rows_v1, sem1)

        return carry

    lax.fori_loop(0, nchunks // 2, body, 0)
    pltpu.sync_copy(pool_v, out_hbm.at[pl.ds(wid * rows_per_w, rows_per_w), :])


def _sc_pool(x_flat, emb_packed, H, B, L, EMB):
    info = plsc.get_sparse_core_info()
    NC, NS = info.num_cores, info.num_subcores
    NW = NC * NS
    rows_per_w = B // NW
    chunk = 4
    mesh = plsc.VectorSubcoreMesh(core_axis_name="c", subcore_axis_name="s")
    body = functools.partial(_pool_body, rows_per_w=rows_per_w, L=L,
                             EMB=EMB, chunk=chunk, NC=NC, H=H)
    return pl.kernel(
        body,
        out_type=jax.ShapeDtypeStruct((B, EMB), jnp.float32),
        mesh=mesh,
        compiler_params=pltpu.CompilerParams(use_tc_tiling_on_sc=True,
                                             needs_layout_passes=False),
        scratch_types=[
            pltpu.VMEM((rows_per_w * L,), jnp.int32),
            pltpu.VMEM((rows_per_w * L,), jnp.int32),
            pltpu.VMEM((chunk * L, 2 * EMB), jnp.float32),
            pltpu.VMEM((chunk * L, 2 * EMB), jnp.float32),
            pltpu.VMEM((rows_per_w, EMB), jnp.float32),
            pltpu.SemaphoreType.DMA,
            pltpu.SemaphoreType.DMA,
        ],
    )(x_flat, emb_packed)


def _mlp_body(p_ref, W1_ref, b1_ref, W2_ref, b2_ref, o_ref):
    h = jnp.dot(p_ref[...], W1_ref[...],
                preferred_element_type=jnp.float32) + b1_ref[...]
    h = jnp.maximum(h, 0.0)
    o_ref[...] = jnp.dot(h, W2_ref[...],
                         preferred_element_type=jnp.float32) + b2_ref[...]


def _tc_mlp(pooled, W1, b1, W2, b2):
    B = pooled.shape[0]
    return pl.pallas_call(
        _mlp_body,
        out_shape=jax.ShapeDtypeStruct((B, W2.shape[1]), jnp.float32),
    )(pooled, W1, b1.reshape(1, -1), W2, b2.reshape(1, -1))


@jax.jit
def kernel(x, emb, W1, b1, W2, b2):
    B, L = x.shape
    EMB = emb.shape[1]
    x_flat = x.reshape(-1).astype(jnp.int32)
    emb_packed, H = _tc_pack(jnp.swapaxes(emb, 0, 1))
    pooled = _sc_pool(x_flat, emb_packed, H, B, L, EMB)
    return _tc_mlp(pooled, W1, b1.reshape(1, -1), W2, b2.reshape(1, -1))


# speedup vs baseline: 2.0601x; 1.0412x over previous
import functools

import jax
import jax.numpy as jnp
from jax import lax
from jax.experimental import pallas as pl
from jax.experimental.pallas import tpu as pltpu
from jax.experimental.pallas import tpu_sc as plsc


def _tr2_body(lo_ref, hi_ref, o_ref):
    o_ref[...] = jnp.concatenate([lo_ref[...].T, hi_ref[...].T], axis=1)


def _tc_pack(embT, blk=16384):
    EMB, V = embT.shape
    nblk_half = (V // 2 + blk - 1) // blk
    H = nblk_half * blk
    max_blk = (V - 1) // blk
    return pl.pallas_call(
        _tr2_body,
        grid=(nblk_half,),
        in_specs=[pl.BlockSpec((EMB, blk), lambda i: (0, i)),
                  pl.BlockSpec((EMB, blk),
                               lambda i: (0, jnp.minimum(i + nblk_half,
                                                         max_blk)))],
        out_specs=pl.BlockSpec((blk, 2 * EMB), lambda i: (i, 0)),
        out_shape=jax.ShapeDtypeStruct((H, 2 * EMB), jnp.float32),
    )(embT, embT), H


def _pool_body(x_hbm, emb_hbm, out_hbm, idx_v, offs_v, rows_v0, rows_v1,
               pool_v, sem0, sem1, *, rows_per_w, L, EMB, chunk, NC, H):
    wid = lax.axis_index("s") * NC + lax.axis_index("c")
    n_idx = rows_per_w * L
    pltpu.sync_copy(x_hbm.at[pl.ds(wid * n_idx, n_idx)], idx_v)

    def prep(k, carry):
        v = idx_v[pl.ds(k * 16, 16)]
        hi = v >= H
        idx_v[pl.ds(k * 16, 16)] = jnp.where(hi, v - H, v)
        offs_v[pl.ds(k * 16, 16)] = jnp.where(hi, EMB, 0).astype(jnp.int32)
        return carry

    lax.fori_loop(0, n_idx // 16, prep, 0)

    cl = chunk * L
    nchunks = rows_per_w // chunk
    nvec = EMB // 16
    iotas = [lax.iota(jnp.int32, 16) + j * 16 for j in range(nvec)]

    def fire(c, buf, sem):
        pltpu.async_copy(emb_hbm.at[idx_v.at[pl.ds(c * cl, cl)]], buf, sem)

    def drain(c, buf, sem):
        pltpu.make_async_copy(
            emb_hbm.at[idx_v.at[pl.ds(c * cl, cl)]], buf, sem).wait()

    def accumulate(c, buf):
        for i in range(chunk):
            acc_a = [jnp.zeros((16,), jnp.float32) for _ in range(nvec)]
            acc_b = [jnp.zeros((16,), jnp.float32) for _ in range(nvec)]
            for l in range(L):
                s = i * L + l
                off = plsc.load_gather(
                    offs_v, [jnp.full((16,), c * cl + s, jnp.int32)])
                row = jnp.full((16,), s, jnp.int32)
                accs = acc_a if l % 2 == 0 else acc_b
                for j in range(nvec):
                    accs[j] = accs[j] + plsc.load_gather(
                        buf, [row, off + iotas[j]])
            for j in range(nvec):
                pool_v[c * chunk + i, pl.ds(j * 16, 16)] = (
                    (acc_a[j] + acc_b[j]) * (1.0 / L))

    fire(0, rows_v0, sem0)
    fire(1, rows_v1, sem1)

    def body(g, carry):
        c0 = 2 * g
        drain(c0, rows_v0, sem0)
        accumulate(c0, rows_v0)

        @pl.when(c0 + 2 < nchunks)
        def _():
            fire(c0 + 2, rows_v0, sem0)

        c1 = c0 + 1
        drain(c1, rows_v1, sem1)
        accumulate(c1, rows_v1)

        @pl.when(c1 + 2 < nchunks)
        def _():
            fire(c1 + 2, rows_v1, sem1)

        return carry

    lax.fori_loop(0, nchunks // 2, body, 0)
    pltpu.sync_copy(pool_v, out_hbm.at[pl.ds(wid * rows_per_w, rows_per_w), :])


def _sc_pool(x_flat, emb_packed, H, B, L, EMB):
    info = plsc.get_sparse_core_info()
    NC, NS = info.num_cores, info.num_subcores
    NW = NC * NS
    rows_per_w = B // NW
    chunk = 4
    mesh = plsc.VectorSubcoreMesh(core_axis_name="c", subcore_axis_name="s")
    body = functools.partial(_pool_body, rows_per_w=rows_per_w, L=L,
                             EMB=EMB, chunk=chunk, NC=NC, H=H)
    return pl.kernel(
        body,
        out_type=jax.ShapeDtypeStruct((B, EMB), jnp.float32),
        mesh=mesh,
        compiler_params=pltpu.CompilerParams(use_tc_tiling_on_sc=True,
                                             needs_layout_passes=False),
        scratch_types=[
            pltpu.VMEM((rows_per_w * L,), jnp.int32),
            pltpu.VMEM((rows_per_w * L,), jnp.int32),
            pltpu.VMEM((chunk * L, 2 * EMB), jnp.float32),
            pltpu.VMEM((chunk * L, 2 * EMB), jnp.float32),
            pltpu.VMEM((rows_per_w, EMB), jnp.float32),
            pltpu.SemaphoreType.DMA,
            pltpu.SemaphoreType.DMA,
        ],
    )(x_flat, emb_packed)


def _mlp_body(p_ref, W1_ref, b1_ref, W2_ref, b2_ref, o_ref):
    h = jnp.dot(p_ref[...], W1_ref[...],
                preferred_element_type=jnp.float32) + b1_ref[...]
    h = jnp.maximum(h, 0.0)
    o_ref[...] = jnp.dot(h, W2_ref[...],
                         preferred_element_type=jnp.float32) + b2_ref[...]


def _tc_mlp(pooled, W1, b1, W2, b2):
    B = pooled.shape[0]
    return pl.pallas_call(
        _mlp_body,
        out_shape=jax.ShapeDtypeStruct((B, W2.shape[1]), jnp.float32),
    )(pooled, W1, b1.reshape(1, -1), W2, b2.reshape(1, -1))


@jax.jit
def kernel(x, emb, W1, b1, W2, b2):
    B, L = x.shape
    EMB = emb.shape[1]
    x_flat = x.reshape(-1).astype(jnp.int32)
    emb_packed, H = _tc_pack(jnp.swapaxes(emb, 0, 1))
    pooled = _sc_pool(x_flat, emb_packed, H, B, L, EMB)
    return _tc_mlp(pooled, W1, b1.reshape(1, -1), W2, b2.reshape(1, -1))


# speedup vs baseline: 2.0644x; 1.0021x over previous
import functools

import jax
import jax.numpy as jnp
from jax import lax
from jax.experimental import pallas as pl
from jax.experimental.pallas import tpu as pltpu
from jax.experimental.pallas import tpu_sc as plsc


def _tr2_body(lo_ref, hi_ref, o_ref):
    o_ref[...] = jnp.concatenate([lo_ref[...].T, hi_ref[...].T], axis=1)


def _tc_pack(embT, blk=16384):
    EMB, V = embT.shape
    nblk_half = (V // 2 + blk - 1) // blk
    H = nblk_half * blk
    max_blk = (V - 1) // blk
    return pl.pallas_call(
        _tr2_body,
        grid=(nblk_half,),
        in_specs=[pl.BlockSpec((EMB, blk), lambda i: (0, i)),
                  pl.BlockSpec((EMB, blk),
                               lambda i: (0, jnp.minimum(i + nblk_half,
                                                         max_blk)))],
        out_specs=pl.BlockSpec((blk, 2 * EMB), lambda i: (i, 0)),
        out_shape=jax.ShapeDtypeStruct((H, 2 * EMB), jnp.float32),
    )(embT, embT), H


def _pool_body(x_hbm, emb_hbm, out_hbm, idx_v, offs_v, rows_v0, rows_v1,
               pool_v, sem0, sem1, *, rows_per_w, L, EMB, chunk, NC, H):
    wid = lax.axis_index("s") * NC + lax.axis_index("c")
    n_idx = rows_per_w * L
    pltpu.sync_copy(x_hbm.at[pl.ds(wid * n_idx, n_idx)], idx_v)

    def prep(k, carry):
        v = idx_v[pl.ds(k * 16, 16)]
        hi = v >= H
        idx_v[pl.ds(k * 16, 16)] = jnp.where(hi, v - H, v)
        offs_v[pl.ds(k * 16, 16)] = jnp.where(hi, EMB, 0).astype(jnp.int32)
        return carry

    lax.fori_loop(0, n_idx // 16, prep, 0)

    cl = chunk * L
    nchunks = rows_per_w // chunk
    nvec = EMB // 16
    iotas = [lax.iota(jnp.int32, 16) + j * 16 for j in range(nvec)]

    def fire(c, buf, sem):
        pltpu.async_copy(emb_hbm.at[idx_v.at[pl.ds(c * cl, cl)]], buf, sem)

    def drain(c, buf, sem):
        pltpu.make_async_copy(
            emb_hbm.at[idx_v.at[pl.ds(c * cl, cl)]], buf, sem).wait()

    def accumulate(c, buf):
        grp = 4
        for i in range(chunk):
            acc_a = [jnp.zeros((16,), jnp.float32) for _ in range(nvec)]
            acc_b = [jnp.zeros((16,), jnp.float32) for _ in range(nvec)]
            for l0 in range(0, L, grp):
                ls = range(l0, min(l0 + grp, L))
                offs = [plsc.load_gather(
                    offs_v,
                    [jnp.full((16,), c * cl + i * L + l, jnp.int32)])
                    for l in ls]
                for t, l in enumerate(ls):
                    row = jnp.full((16,), i * L + l, jnp.int32)
                    accs = acc_a if l % 2 == 0 else acc_b
                    for j in range(nvec):
                        accs[j] = accs[j] + plsc.load_gather(
                            buf, [row, offs[t] + iotas[j]])
            for j in range(nvec):
                pool_v[c * chunk + i, pl.ds(j * 16, 16)] = (
                    (acc_a[j] + acc_b[j]) * (1.0 / L))

    fire(0, rows_v0, sem0)
    fire(1, rows_v1, sem1)

    def body(g, carry):
        c0 = 2 * g
        drain(c0, rows_v0, sem0)
        accumulate(c0, rows_v0)

        @pl.when(c0 + 2 < nchunks)
        def _():
            fire(c0 + 2, rows_v0, sem0)

        c1 = c0 + 1
        drain(c1, rows_v1, sem1)
        accumulate(c1, rows_v1)

        @pl.when(c1 + 2 < nchunks)
        def _():
            fire(c1 + 2, rows_v1, sem1)

        return carry

    lax.fori_loop(0, nchunks // 2, body, 0)
    pltpu.sync_copy(pool_v, out_hbm.at[pl.ds(wid * rows_per_w, rows_per_w), :])


def _sc_pool(x_flat, emb_packed, H, B, L, EMB):
    info = plsc.get_sparse_core_info()
    NC, NS = info.num_cores, info.num_subcores
    NW = NC * NS
    rows_per_w = B // NW
    chunk = 4
    mesh = plsc.VectorSubcoreMesh(core_axis_name="c", subcore_axis_name="s")
    body = functools.partial(_pool_body, rows_per_w=rows_per_w, L=L,
                             EMB=EMB, chunk=chunk, NC=NC, H=H)
    return pl.kernel(
        body,
        out_type=jax.ShapeDtypeStruct((B, EMB), jnp.float32),
        mesh=mesh,
        compiler_params=pltpu.CompilerParams(use_tc_tiling_on_sc=True,
                                             needs_layout_passes=False),
        scratch_types=[
            pltpu.VMEM((rows_per_w * L,), jnp.int32),
            pltpu.VMEM((rows_per_w * L,), jnp.int32),
            pltpu.VMEM((chunk * L, 2 * EMB), jnp.float32),
            pltpu.VMEM((chunk * L, 2 * EMB), jnp.float32),
            pltpu.VMEM((rows_per_w, EMB), jnp.float32),
            pltpu.SemaphoreType.DMA,
            pltpu.SemaphoreType.DMA,
        ],
    )(x_flat, emb_packed)


def _mlp_body(p_ref, W1_ref, b1_ref, W2_ref, b2_ref, o_ref):
    h = jnp.dot(p_ref[...], W1_ref[...],
                preferred_element_type=jnp.float32) + b1_ref[...]
    h = jnp.maximum(h, 0.0)
    o_ref[...] = jnp.dot(h, W2_ref[...],
                         preferred_element_type=jnp.float32) + b2_ref[...]


def _tc_mlp(pooled, W1, b1, W2, b2):
    B = pooled.shape[0]
    return pl.pallas_call(
        _mlp_body,
        out_shape=jax.ShapeDtypeStruct((B, W2.shape[1]), jnp.float32),
    )(pooled, W1, b1.reshape(1, -1), W2, b2.reshape(1, -1))


@jax.jit
def kernel(x, emb, W1, b1, W2, b2):
    B, L = x.shape
    EMB = emb.shape[1]
    x_flat = x.reshape(-1).astype(jnp.int32)
    emb_packed, H = _tc_pack(jnp.swapaxes(emb, 0, 1))
    pooled = _sc_pool(x_flat, emb_packed, H, B, L, EMB)
    return _tc_mlp(pooled, W1, b1.reshape(1, -1), W2, b2.reshape(1, -1))
